# baseline (speedup 1.0000x reference)
import functools

import jax
import jax.numpy as jnp
from jax import lax
from jax.experimental import pallas as pl
from jax.experimental.pallas import tpu as pltpu
from jax.experimental.pallas import tpu_sc as plsc

N_TOK = 16384
K_CODES = 8192
D = 256
BETA = 0.25

T = 256
KC = 2048
N_TILES = N_TOK // T
N_KCH = K_CODES // KC
WIN = (0, 2736, 5472, 8192)

SC_CORES = 2
SC_SUBCORES = 16
SC_WORKERS = SC_CORES * SC_SUBCORES
TOK_PER_W = N_TOK // SC_WORKERS
GATHER_CHUNK = 128
CHUNKS_PER_W = TOK_PER_W // GATHER_CHUNK


def _vq_argmin_body(zf_ref, zn_ref, w_ref, wn_ref,
                    idx_ref, loss_ref, acc_ref):
    i = pl.program_id(0)

    @pl.when(i == 0)
    def _init():
        acc_ref[...] = jnp.zeros_like(acc_ref)

    zf = zf_ref[...]
    zn = zn_ref[...]
    BIG = jnp.int32(2 ** 30)
    INF = jnp.float32(jnp.inf)

    wm = [None, None, None]
    wa = [None, None, None]
    for c in range(N_KCH):
        w_c = w_ref[pl.ds(c * KC, KC), :]
        dot = lax.dot_general(zf, w_c, (((1,), (1,)), ((), ())),
                              preferred_element_type=jnp.float32)
        wn_c = wn_ref[:, pl.ds(c * KC, KC)]
        d_c = (zn + wn_c) - 2.0 * dot
        ii = lax.broadcasted_iota(jnp.int32, (T, KC), 1) + jnp.int32(c * KC)
        for w in range(3):
            lo, hi = WIN[w], WIN[w + 1]
            if c * KC >= hi or (c + 1) * KC <= lo:
                continue
            if lo <= c * KC and (c + 1) * KC <= hi:
                dw = d_c
            else:
                dw = jnp.where((ii >= lo) & (ii < hi), d_c, INF)
            m_cw = jnp.min(dw, axis=1, keepdims=True)
            cand = jnp.where(dw == m_cw, ii, BIG)
            a_cw = jnp.min(cand, axis=1, keepdims=True)
            if wm[w] is None:
                wm[w], wa[w] = m_cw, a_cw
            else:
                upd = m_cw < wm[w]
                wa[w] = jnp.where(upd, a_cw, wa[w])
                wm[w] = jnp.where(upd, m_cw, wm[w])

    accv = wm[0].astype(jnp.bfloat16).astype(jnp.float32)
    a = wa[0]
    for w in (1, 2):
        upd = wm[w] < accv
        a = jnp.where(upd, wa[w], a)
        accv = jnp.where(upd, wm[w], accv).astype(jnp.bfloat16).astype(jnp.float32)

    m = jnp.minimum(jnp.minimum(wm[0], wm[1]), wm[2])
    idx_ref[...] = a
    acc_ref[...] += jnp.sum(m, axis=0, keepdims=True)

    @pl.when(i == N_TILES - 1)
    def _fini():
        loss_ref[...] = acc_ref[...] * jnp.float32(BETA / (N_TOK * D))


_vq_argmin = pl.pallas_call(
    _vq_argmin_body,
    grid=(N_TILES,),
    in_specs=[
        pl.BlockSpec((T, D), lambda i: (i, 0)),
        pl.BlockSpec((T, 1), lambda i: (i, 0)),
        pl.BlockSpec((K_CODES, D), lambda i: (0, 0)),
        pl.BlockSpec((1, K_CODES), lambda i: (0, 0)),
    ],
    out_specs=[
        pl.BlockSpec((T, 1), lambda i: (i, 0)),
        pl.BlockSpec((1, 1), lambda i: (0, 0)),
    ],
    out_shape=[
        jax.ShapeDtypeStruct((N_TOK, 1), jnp.int32),
        jax.ShapeDtypeStruct((1, 1), jnp.float32),
    ],
    scratch_shapes=[
        pltpu.VMEM((1, 1), jnp.float32),
    ],
)


def _vq_onehot_body(idx_ref, enc_ref, perp_ref, cnt_ref):
    i = pl.program_id(0)

    @pl.when(i == 0)
    def _init():
        cnt_ref[...] = jnp.zeros_like(cnt_ref)

    a = idx_ref[...]
    for c in range(N_KCH):
        ii = lax.broadcasted_iota(jnp.int32, (T, KC), 1) + jnp.int32(c * KC)
        enc_c = (ii == a).astype(jnp.float32)
        enc_ref[:, pl.ds(c * KC, KC)] = enc_c
        cnt_ref[:, pl.ds(c * KC, KC)] += jnp.sum(enc_c, axis=0, keepdims=True)

    @pl.when(i == N_TILES - 1)
    def _fini():
        p = cnt_ref[...] * jnp.float32(1.0 / N_TOK)
        ent = jnp.sum(p * jnp.log(p + 1e-10), axis=1, keepdims=True)
        perp_ref[...] = jnp.exp(-ent)


_vq_onehot = pl.pallas_call(
    _vq_onehot_body,
    grid=(N_TILES,),
    in_specs=[
        pl.BlockSpec((T, 1), lambda i: (i, 0)),
    ],
    out_specs=[
        pl.BlockSpec((T, K_CODES), lambda i: (i, 0)),
        pl.BlockSpec((1, 1), lambda i: (0, 0)),
    ],
    out_shape=[
        jax.ShapeDtypeStruct((N_TOK, K_CODES), jnp.float32),
        jax.ShapeDtypeStruct((1, 1), jnp.float32),
    ],
    scratch_shapes=[
        pltpu.VMEM((1, K_CODES), jnp.float32),
    ],
)


def _sc_gather_body(w_hbm, idx_hbm, out_hbm, idx_v, rows_v, sem):
    c = lax.axis_index("c")
    s = lax.axis_index("s")
    wid = s * SC_CORES + c
    pltpu.sync_copy(idx_hbm.at[pl.ds(wid * CHUNKS_PER_W, CHUNKS_PER_W)], idx_v)
    for j in range(CHUNKS_PER_W):
        pltpu.async_copy(w_hbm.at[idx_v.at[j]], rows_v, sem).wait()
        base = wid * TOK_PER_W + j * GATHER_CHUNK
        pltpu.sync_copy(rows_v, out_hbm.at[pl.ds(base, GATHER_CHUNK)])


@functools.lru_cache(maxsize=1)
def _make_sc_gather():
    return pl.kernel(
        _sc_gather_body,
        out_type=jax.ShapeDtypeStruct((N_TOK, D), jnp.float32),
        mesh=plsc.VectorSubcoreMesh(core_axis_name="c", subcore_axis_name="s"),
        scratch_types=[
            pltpu.VMEM((CHUNKS_PER_W, GATHER_CHUNK), jnp.int32),
            pltpu.VMEM((GATHER_CHUNK, D), jnp.float32),
            pltpu.SemaphoreType.DMA,
        ],
    )


def kernel(z, weight):
    zp = jnp.transpose(z, (0, 2, 1))
    zf = zp.reshape(-1, D)
    zn = (zf ** 2).sum(axis=1, keepdims=True)
    wn = (weight ** 2).sum(axis=1).reshape(1, K_CODES)

    idx2, loss = _vq_argmin(zf, zn, weight, wn)
    idx = idx2.reshape(-1)

    enc, perp = _vq_onehot(idx2)
    zq_rows = _make_sc_gather()(weight, idx2.reshape(-1, GATHER_CHUNK))
    z_q_out = jnp.transpose(zq_rows.reshape(z.shape[0], -1, D), (0, 2, 1))

    return (z_q_out, (perp.reshape(()), enc, idx), loss.reshape(()))


# speedup vs baseline: 1.2038x; 1.2038x over previous
import functools

import jax
import jax.numpy as jnp
from jax import lax
from jax.experimental import pallas as pl
from jax.experimental.pallas import tpu as pltpu
from jax.experimental.pallas import tpu_sc as plsc

N_TOK = 16384
K_CODES = 8192
D = 256
BETA = 0.25

T = 256
KC = 2048
N_TILES = N_TOK // T
N_KCH = K_CODES // KC

SC_CORES = 2
SC_SUBCORES = 16
SC_WORKERS = SC_CORES * SC_SUBCORES
TOK_PER_W = N_TOK // SC_WORKERS
GATHER_CHUNK = 128
CHUNKS_PER_W = TOK_PER_W // GATHER_CHUNK


WIN = (0, 2736, 5472, 8192)


def _vq_body(zf_ref, zn_ref, w_ref, wn_ref,
             idx_ref, enc_ref, perp_ref, loss_ref,
             cnt_ref, acc_ref):
    i = pl.program_id(0)

    @pl.when(i == 0)
    def _init():
        cnt_ref[...] = jnp.zeros_like(cnt_ref)
        acc_ref[...] = jnp.zeros_like(acc_ref)

    zf = zf_ref[...]
    zn = zn_ref[...]
    BIG = jnp.int32(2 ** 30)
    INF = jnp.float32(jnp.inf)

    wm = [None, None, None]
    wa = [None, None, None]
    for c in range(N_KCH):
        w_c = w_ref[pl.ds(c * KC, KC), :]
        dot = lax.dot_general(zf, w_c, (((1,), (1,)), ((), ())),
                              preferred_element_type=jnp.float32)
        wn_c = wn_ref[:, pl.ds(c * KC, KC)]
        d_c = (zn + wn_c) - 2.0 * dot
        ii = lax.broadcasted_iota(jnp.int32, (T, KC), 1) + jnp.int32(c * KC)
        for w in range(3):
            lo, hi = WIN[w], WIN[w + 1]
            if c * KC >= hi or (c + 1) * KC <= lo:
                continue
            if lo <= c * KC and (c + 1) * KC <= hi:
                dw = d_c
            else:
                dw = jnp.where((ii >= lo) & (ii < hi), d_c, INF)
            m_cw = jnp.min(dw, axis=1, keepdims=True)
            cand = jnp.where(dw == m_cw, ii, BIG)
            a_cw = jnp.min(cand, axis=1, keepdims=True)
            if wm[w] is None:
                wm[w], wa[w] = m_cw, a_cw
            else:
                upd = m_cw < wm[w]
                wa[w] = jnp.where(upd, a_cw, wa[w])
                wm[w] = jnp.where(upd, m_cw, wm[w])

    accv = wm[0].astype(jnp.bfloat16).astype(jnp.float32)
    a = wa[0]
    for w in (1, 2):
        upd = wm[w] < accv
        a = jnp.where(upd, wa[w], a)
        accv = jnp.where(upd, wm[w], accv).astype(jnp.bfloat16).astype(jnp.float32)

    m = jnp.minimum(jnp.minimum(wm[0], wm[1]), wm[2])
    idx_ref[...] = a
    acc_ref[...] += jnp.sum(m, axis=0, keepdims=True)

    for c in range(N_KCH):
        ii = lax.broadcasted_iota(jnp.int32, (T, KC), 1) + jnp.int32(c * KC)
        enc_c = (ii == a).astype(jnp.float32)
        enc_ref[:, pl.ds(c * KC, KC)] = enc_c
        cnt_ref[:, pl.ds(c * KC, KC)] += jnp.sum(enc_c, axis=0, keepdims=True)

    @pl.when(i == N_TILES - 1)
    def _fini():
        p = cnt_ref[...] * jnp.float32(1.0 / N_TOK)
        ent = jnp.sum(p * jnp.log(p + 1e-10), axis=1, keepdims=True)
        perp_ref[...] = jnp.exp(-ent)
        loss_ref[...] = acc_ref[...] * jnp.float32(BETA / (N_TOK * D))


_vq_main = pl.pallas_call(
    _vq_body,
    grid=(N_TILES,),
    in_specs=[
        pl.BlockSpec((T, D), lambda i: (i, 0)),
        pl.BlockSpec((T, 1), lambda i: (i, 0)),
        pl.BlockSpec((K_CODES, D), lambda i: (0, 0)),
        pl.BlockSpec((1, K_CODES), lambda i: (0, 0)),
    ],
    out_specs=[
        pl.BlockSpec((T, 1), lambda i: (i, 0)),
        pl.BlockSpec((T, K_CODES), lambda i: (i, 0)),
        pl.BlockSpec((1, 1), lambda i: (0, 0)),
        pl.BlockSpec((1, 1), lambda i: (0, 0)),
    ],
    out_shape=[
        jax.ShapeDtypeStruct((N_TOK, 1), jnp.int32),
        jax.ShapeDtypeStruct((N_TOK, K_CODES), jnp.float32),
        jax.ShapeDtypeStruct((1, 1), jnp.float32),
        jax.ShapeDtypeStruct((1, 1), jnp.float32),
    ],
    scratch_shapes=[
        pltpu.VMEM((1, K_CODES), jnp.float32),
        pltpu.VMEM((1, 1), jnp.float32),
    ],
)


def _sc_gather_body(w_hbm, idx_hbm, out_hbm, idx_v, rows_v, sem):
    c = lax.axis_index("c")
    s = lax.axis_index("s")
    wid = s * SC_CORES + c
    pltpu.sync_copy(idx_hbm.at[pl.ds(wid * CHUNKS_PER_W, CHUNKS_PER_W)], idx_v)
    for j in range(CHUNKS_PER_W):
        pltpu.async_copy(w_hbm.at[idx_v.at[j]], rows_v, sem).wait()
        base = wid * TOK_PER_W + j * GATHER_CHUNK
        pltpu.sync_copy(rows_v, out_hbm.at[pl.ds(base, GATHER_CHUNK)])


@functools.lru_cache(maxsize=1)
def _make_sc_gather():
    return pl.kernel(
        _sc_gather_body,
        out_type=jax.ShapeDtypeStruct((N_TOK, D), jnp.float32),
        mesh=plsc.VectorSubcoreMesh(core_axis_name="c", subcore_axis_name="s"),
        scratch_types=[
            pltpu.VMEM((CHUNKS_PER_W, GATHER_CHUNK), jnp.int32),
            pltpu.VMEM((GATHER_CHUNK, D), jnp.float32),
            pltpu.SemaphoreType.DMA,
        ],
    )


def kernel(z, weight):
    zp = jnp.transpose(z, (0, 2, 1))
    zf = zp.reshape(-1, D)
    zn = (zf ** 2).sum(axis=1, keepdims=True)
    wn = (weight ** 2).sum(axis=1).reshape(1, K_CODES)

    idx2, enc, perp, loss = _vq_main(zf, zn, weight, wn)
    idx = idx2.reshape(-1)

    zq_rows = _make_sc_gather()(weight, idx2.reshape(-1, GATHER_CHUNK))
    z_q_out = jnp.transpose(zq_rows.reshape(z.shape[0], -1, D), (0, 2, 1))

    return (z_q_out, (perp.reshape(()), enc, idx), loss.reshape(()))


# speedup vs baseline: 1.2644x; 1.0503x over previous
import functools

import jax
import jax.numpy as jnp
from jax import lax
from jax.experimental import pallas as pl
from jax.experimental.pallas import tpu as pltpu
from jax.experimental.pallas import tpu_sc as plsc

N_TOK = 16384
K_CODES = 8192
D = 256
BETA = 0.25

T = 512
KC = 2048
N_TILES = N_TOK // T
N_KCH = K_CODES // KC

SC_CORES = 2
SC_SUBCORES = 16
SC_WORKERS = SC_CORES * SC_SUBCORES
TOK_PER_W = N_TOK // SC_WORKERS
GATHER_CHUNK = 128
CHUNKS_PER_W = TOK_PER_W // GATHER_CHUNK


WIN = (0, 2736, 5472, 8192)


def _vq_body(zf_ref, zn_ref, w_ref, wn_ref,
             idx_ref, enc_ref, perp_ref, loss_ref,
             cnt_ref, acc_ref):
    i = pl.program_id(0)

    @pl.when(i == 0)
    def _init():
        cnt_ref[...] = jnp.zeros_like(cnt_ref)
        acc_ref[...] = jnp.zeros_like(acc_ref)

    zf = zf_ref[...]
    zn = zn_ref[...]
    BIG = jnp.int32(2 ** 30)
    INF = jnp.float32(jnp.inf)

    wm = [None, None, None]
    wa = [None, None, None]
    for c in range(N_KCH):
        w_c = w_ref[pl.ds(c * KC, KC), :]
        dot = lax.dot_general(zf, w_c, (((1,), (1,)), ((), ())),
                              preferred_element_type=jnp.float32)
        wn_c = wn_ref[:, pl.ds(c * KC, KC)]
        d_c = (zn + wn_c) - 2.0 * dot
        ii = lax.broadcasted_iota(jnp.int32, (T, KC), 1) + jnp.int32(c * KC)
        for w in range(3):
            lo, hi = WIN[w], WIN[w + 1]
            if c * KC >= hi or (c + 1) * KC <= lo:
                continue
            if lo <= c * KC and (c + 1) * KC <= hi:
                dw = d_c
            else:
                dw = jnp.where((ii >= lo) & (ii < hi), d_c, INF)
            m_cw = jnp.min(dw, axis=1, keepdims=True)
            cand = jnp.where(dw == m_cw, ii, BIG)
            a_cw = jnp.min(cand, axis=1, keepdims=True)
            if wm[w] is None:
                wm[w], wa[w] = m_cw, a_cw
            else:
                upd = m_cw < wm[w]
                wa[w] = jnp.where(upd, a_cw, wa[w])
                wm[w] = jnp.where(upd, m_cw, wm[w])

    accv = wm[0].astype(jnp.bfloat16).astype(jnp.float32)
    a = wa[0]
    for w in (1, 2):
        upd = wm[w] < accv
        a = jnp.where(upd, wa[w], a)
        accv = jnp.where(upd, wm[w], accv).astype(jnp.bfloat16).astype(jnp.float32)

    m = jnp.minimum(jnp.minimum(wm[0], wm[1]), wm[2])
    idx_ref[...] = a
    acc_ref[...] += jnp.sum(m, axis=0, keepdims=True)

    for c in range(N_KCH):
        ii = lax.broadcasted_iota(jnp.int32, (T, KC), 1) + jnp.int32(c * KC)
        enc_c = (ii == a).astype(jnp.float32)
        enc_ref[:, pl.ds(c * KC, KC)] = enc_c
        cnt_ref[:, pl.ds(c * KC, KC)] += jnp.sum(enc_c, axis=0, keepdims=True)

    @pl.when(i == N_TILES - 1)
    def _fini():
        p = cnt_ref[...] * jnp.float32(1.0 / N_TOK)
        ent = jnp.sum(p * jnp.log(p + 1e-10), axis=1, keepdims=True)
        perp_ref[...] = jnp.exp(-ent)
        loss_ref[...] = acc_ref[...] * jnp.float32(BETA / (N_TOK * D))


_vq_main = pl.pallas_call(
    _vq_body,
    grid=(N_TILES,),
    in_specs=[
        pl.BlockSpec((T, D), lambda i: (i, 0)),
        pl.BlockSpec((T, 1), lambda i: (i, 0)),
        pl.BlockSpec((K_CODES, D), lambda i: (0, 0)),
        pl.BlockSpec((1, K_CODES), lambda i: (0, 0)),
    ],
    out_specs=[
        pl.BlockSpec((T, 1), lambda i: (i, 0)),
        pl.BlockSpec((T, K_CODES), lambda i: (i, 0)),
        pl.BlockSpec((1, 1), lambda i: (0, 0)),
        pl.BlockSpec((1, 1), lambda i: (0, 0)),
    ],
    out_shape=[
        jax.ShapeDtypeStruct((N_TOK, 1), jnp.int32),
        jax.ShapeDtypeStruct((N_TOK, K_CODES), jnp.float32),
        jax.ShapeDtypeStruct((1, 1), jnp.float32),
        jax.ShapeDtypeStruct((1, 1), jnp.float32),
    ],
    scratch_shapes=[
        pltpu.VMEM((1, K_CODES), jnp.float32),
        pltpu.VMEM((1, 1), jnp.float32),
    ],
)


def _sc_gather_body(w_hbm, idx_hbm, out_hbm, idx_v, rows_v, sem):
    c = lax.axis_index("c")
    s = lax.axis_index("s")
    wid = s * SC_CORES + c
    pltpu.sync_copy(idx_hbm.at[pl.ds(wid * CHUNKS_PER_W, CHUNKS_PER_W)], idx_v)
    for j in range(CHUNKS_PER_W):
        pltpu.async_copy(w_hbm.at[idx_v.at[j]], rows_v, sem).wait()
        base = wid * TOK_PER_W + j * GATHER_CHUNK
        pltpu.sync_copy(rows_v, out_hbm.at[pl.ds(base, GATHER_CHUNK)])


@functools.lru_cache(maxsize=1)
def _make_sc_gather():
    return pl.kernel(
        _sc_gather_body,
        out_type=jax.ShapeDtypeStruct((N_TOK, D), jnp.float32),
        mesh=plsc.VectorSubcoreMesh(core_axis_name="c", subcore_axis_name="s"),
        scratch_types=[
            pltpu.VMEM((CHUNKS_PER_W, GATHER_CHUNK), jnp.int32),
            pltpu.VMEM((GATHER_CHUNK, D), jnp.float32),
            pltpu.SemaphoreType.DMA,
        ],
    )


def kernel(z, weight):
    zp = jnp.transpose(z, (0, 2, 1))
    zf = zp.reshape(-1, D)
    zn = (zf ** 2).sum(axis=1, keepdims=True)
    wn = (weight ** 2).sum(axis=1).reshape(1, K_CODES)

    idx2, enc, perp, loss = _vq_main(zf, zn, weight, wn)
    idx = idx2.reshape(-1)

    zq_rows = _make_sc_gather()(weight, idx2.reshape(-1, GATHER_CHUNK))
    z_q_out = jnp.transpose(zq_rows.reshape(z.shape[0], -1, D), (0, 2, 1))

    return (z_q_out, (perp.reshape(()), enc, idx), loss.reshape(()))
